# baseline (speedup 1.0000x reference)
import jax
import jax.numpy as jnp
import numpy as np
from jax.experimental import pallas as pl
from jax.experimental.pallas import tpu as pltpu

N = 512
HID = 64
NUM_SPECIES = 10
TDIM = 32
NBASIS = 8
NINTER = 2
R_MAX = 5.0
AVG_NEIGH = 511.0

_MUS = [b * (R_MAX / (NBASIS - 1)) for b in range(NBASIS)]


def _mace_body(pos_ref, nf_ref, gf_ref, se_ref, Wt_ref, Wrad_ref,
               Wupd_ref, Wo1_ref, Wo2_ref, fs_ref, out_ref, rb_ref):
    f32 = jnp.float32
    pos = pos_ref[:, :]

    gram = jax.lax.dot_general(pos, pos, (((1,), (1,)), ((), ())),
                               precision=jax.lax.Precision.HIGHEST,
                               preferred_element_type=f32)
    n2c = jnp.sum(pos * pos, axis=1, keepdims=True)
    n2r = jnp.transpose(n2c)
    d2 = jnp.maximum(n2c + (n2r - 2.0 * gram), 0.0)
    d2p = d2 + 1e-12
    rinv = jax.lax.rsqrt(d2p)
    L = d2p * rinv

    m = jnp.minimum(L * (1.0 / R_MAX), 1.0)
    t = (m - 0.5) * jnp.pi
    t2 = t * t
    p = 1.0 / 362880.0
    p = p * t2 - 1.0 / 5040.0
    p = p * t2 + 1.0 / 120.0
    p = p * t2 - 1.0 / 6.0
    p = p * t2 + 1.0
    env = 0.5 * (1.0 - t * p)
    row = jax.lax.broadcasted_iota(jnp.int32, (N, N), 0)
    col = jax.lax.broadcasted_iota(jnp.int32, (N, N), 1)
    offdiag = jnp.where(row == col, jnp.zeros((), f32), jnp.ones((), f32))
    enm = env * offdiag

    step = R_MAX / (NBASIS - 1)
    Lc = m * R_MAX
    base = jnp.exp(-(Lc * Lc)) * enm
    E = jnp.exp(Lc * (2.0 * step))
    Eb = base
    for b in range(NBASIS):
        rb_ref[b * N:(b + 1) * N, :] = Eb.astype(jnp.bfloat16)
        if b + 1 < NBASIS:
            Eb = Eb * E

    iota_sp = jax.lax.broadcasted_iota(jnp.int32, (N, NUM_SPECIES), 1)
    onehot = (nf_ref[:, :] == iota_sp).astype(f32)
    h = (jnp.dot(onehot, se_ref[:, :], preferred_element_type=f32)
         + jnp.dot(gf_ref[:, :], Wt_ref[:, :], preferred_element_type=f32))

    for t in range(NINTER):
        acc = jnp.zeros((N, HID), f32)
        for b in range(NBASIS):
            cb = float(np.exp(-((b * step) ** 2)))
            rbb = rb_ref[b * N:(b + 1) * N, :]
            hb = (h * (Wrad_ref[t * NBASIS + b:t * NBASIS + b + 1, :] * cb)
                  ).astype(jnp.bfloat16)
            acc = acc + jnp.dot(rbb, hb, preferred_element_type=f32)
        agg = acc * (1.0 / AVG_NEIGH)
        h = h + jnp.tanh(jnp.dot(agg, Wupd_ref[t * HID:(t + 1) * HID, :],
                                 preferred_element_type=f32))

    scal = jnp.dot(jnp.tanh(jnp.dot(h, Wo1_ref[:, :], preferred_element_type=f32)),
                   Wo2_ref[:, :], preferred_element_type=f32)
    Wmat = scal * offdiag * rinv
    pos4 = jnp.concatenate([pos, jnp.ones((N, 1), f32)], axis=1)
    out4 = jax.lax.dot_general(Wmat, pos4, (((0,), (0,)), ((), ())),
                               preferred_element_type=f32)
    vec = pos * out4[:, 3:4] - out4[:, 0:3]
    out_ref[:, :] = vec * (fs_ref[0, 0] * (1.0 / AVG_NEIGH))


def kernel(positions, node_features, global_features, species_embed, W_time,
           W_rad, W_upd, W_out1, W_out2, final_scaling):
    pos = positions.astype(jnp.float32)
    nf = node_features.astype(jnp.int32).reshape(N, 1)
    gf = global_features.astype(jnp.float32).reshape(1, TDIM)
    Wrad2 = W_rad.astype(jnp.float32).reshape(NINTER * NBASIS, HID)
    Wupd2 = W_upd.astype(jnp.float32).reshape(NINTER * HID, HID)
    fs = final_scaling.astype(jnp.float32).reshape(1, 1)

    vmem = pl.BlockSpec(memory_space=pltpu.VMEM)
    return pl.pallas_call(
        _mace_body,
        out_shape=jax.ShapeDtypeStruct((N, 3), jnp.float32),
        in_specs=[vmem] * 10,
        out_specs=vmem,
        scratch_shapes=[pltpu.VMEM((NBASIS * N, N), jnp.bfloat16)],
    )(pos, nf, gf, species_embed.astype(jnp.float32),
      W_time.astype(jnp.float32), Wrad2, Wupd2,
      W_out1.astype(jnp.float32), W_out2.astype(jnp.float32), fs)


# speedup vs baseline: 1.1612x; 1.1612x over previous
import jax
import jax.numpy as jnp
import numpy as np
from jax.experimental import pallas as pl
from jax.experimental.pallas import tpu as pltpu

N = 512
HID = 64
NUM_SPECIES = 10
TDIM = 32
NBASIS = 8
NINTER = 2
R_MAX = 5.0
AVG_NEIGH = 511.0

_MUS = [b * (R_MAX / (NBASIS - 1)) for b in range(NBASIS)]


def _mace_body(pos_ref, posT_ref, nf_ref, gf_ref, se_ref, Wt_ref, Wrad_ref,
               Wupd_ref, Wo1_ref, Wo2_ref, fs_ref, out_ref, rb_ref):
    f32 = jnp.float32
    pos = pos_ref[:, :]

    dk0 = posT_ref[0:1, :] - pos[:, 0:1]
    dk1 = posT_ref[1:2, :] - pos[:, 1:2]
    dk2 = posT_ref[2:3, :] - pos[:, 2:3]
    d2 = dk0 * dk0 + dk1 * dk1 + dk2 * dk2
    d2p = d2 + 1e-12
    rinv = jax.lax.rsqrt(d2p)
    L = d2p * rinv

    t = (jnp.minimum(L * (1.0 / R_MAX), 1.0) - 0.5) * jnp.pi
    t2 = t * t
    p = 1.0 / 362880.0
    p = p * t2 - 1.0 / 5040.0
    p = p * t2 + 1.0 / 120.0
    p = p * t2 - 1.0 / 6.0
    p = p * t2 + 1.0
    env = 0.5 * (1.0 - t * p)
    row = jax.lax.broadcasted_iota(jnp.int32, (N, N), 0)
    col = jax.lax.broadcasted_iota(jnp.int32, (N, N), 1)
    offdiag = jnp.where(row == col, jnp.zeros((), f32), jnp.ones((), f32))
    enm = env * offdiag

    iota_sp = jax.lax.broadcasted_iota(jnp.int32, (N, NUM_SPECIES), 1)
    onehot = (nf_ref[:, :] == iota_sp).astype(f32)
    h = (jnp.dot(onehot, se_ref[:, :], preferred_element_type=f32)
         + jnp.dot(gf_ref[:, :], Wt_ref[:, :], preferred_element_type=f32))

    f8 = jnp.float8_e4m3fn
    inv_scale = 1.0 / (AVG_NEIGH * 256.0 * 64.0)
    acc = jnp.zeros((N, HID), f32)
    for b in range(NBASIS):
        rbb = (jnp.exp(-((L - _MUS[b]) ** 2)) * (enm * 256.0)).astype(f8)
        rb_ref[b * N:(b + 1) * N, :] = rbb
        hb = (h * (Wrad_ref[b:b + 1, :] * 64.0)).astype(f8)
        acc = acc + jnp.dot(rbb, hb, preferred_element_type=f32)
    h = h + jnp.tanh(jnp.dot(acc * inv_scale,
                             Wupd_ref[0:HID, :], preferred_element_type=f32))

    acc = jnp.zeros((N, HID), f32)
    for b in range(NBASIS):
        rbb = rb_ref[b * N:(b + 1) * N, :]
        hb = (h * (Wrad_ref[NBASIS + b:NBASIS + b + 1, :] * 64.0)).astype(f8)
        acc = acc + jnp.dot(rbb, hb, preferred_element_type=f32)
    h = h + jnp.tanh(jnp.dot(acc * inv_scale,
                             Wupd_ref[HID:2 * HID, :], preferred_element_type=f32))

    scal = jnp.dot(jnp.tanh(jnp.dot(h, Wo1_ref[:, :], preferred_element_type=f32)),
                   Wo2_ref[:, :], preferred_element_type=f32)
    Wmat = scal * offdiag * rinv
    pos4 = jnp.concatenate([pos, jnp.ones((N, 1), f32)], axis=1)
    out4 = jax.lax.dot_general(Wmat, pos4, (((0,), (0,)), ((), ())),
                               preferred_element_type=f32)
    vec = pos * out4[:, 3:4] - out4[:, 0:3]
    out_ref[:, :] = vec * (fs_ref[0, 0] * (1.0 / AVG_NEIGH))


def kernel(positions, node_features, global_features, species_embed, W_time,
           W_rad, W_upd, W_out1, W_out2, final_scaling):
    pos = positions.astype(jnp.float32)
    posT = pos.T
    nf = node_features.astype(jnp.int32).reshape(N, 1)
    gf = global_features.astype(jnp.float32).reshape(1, TDIM)
    Wrad2 = W_rad.astype(jnp.float32).reshape(NINTER * NBASIS, HID)
    Wupd2 = W_upd.astype(jnp.float32).reshape(NINTER * HID, HID)
    fs = final_scaling.astype(jnp.float32).reshape(1, 1)

    vmem = pl.BlockSpec(memory_space=pltpu.VMEM)
    return pl.pallas_call(
        _mace_body,
        out_shape=jax.ShapeDtypeStruct((N, 3), jnp.float32),
        in_specs=[vmem] * 11,
        out_specs=vmem,
        scratch_shapes=[pltpu.VMEM((NBASIS * N, N), jnp.float8_e4m3fn)],
    )(pos, posT, nf, gf, species_embed.astype(jnp.float32),
      W_time.astype(jnp.float32), Wrad2, Wupd2,
      W_out1.astype(jnp.float32), W_out2.astype(jnp.float32), fs)


# speedup vs baseline: 1.2355x; 1.0640x over previous
import jax
import jax.numpy as jnp
import numpy as np
from jax.experimental import pallas as pl
from jax.experimental.pallas import tpu as pltpu

N = 512
HID = 64
NUM_SPECIES = 10
TDIM = 32
NBASIS = 8
NINTER = 2
R_MAX = 5.0
AVG_NEIGH = 511.0

_MUS = [b * (R_MAX / (NBASIS - 1)) for b in range(NBASIS)]


def _mace_body(pos_ref, posT_ref, nf_ref, gf_ref, se_ref, Wt_ref, Wrad_ref,
               Wupd_ref, Wo1_ref, Wo2_ref, fs_ref, out_ref, rb_ref):
    f32 = jnp.float32
    pos = pos_ref[:, :]

    dk0 = posT_ref[0:1, :] - pos[:, 0:1]
    dk1 = posT_ref[1:2, :] - pos[:, 1:2]
    dk2 = posT_ref[2:3, :] - pos[:, 2:3]
    d2 = dk0 * dk0 + dk1 * dk1 + dk2 * dk2
    d2p = d2 + 1e-12
    rinv = jax.lax.rsqrt(d2p)
    L = d2p * rinv

    t = (jnp.minimum(L * (1.0 / R_MAX), 1.0) - 0.5) * jnp.pi
    t2 = t * t
    p = -1.8447207e-4
    p = p * t2 + 8.3095165e-3
    p = p * t2 - 1.6665168e-1
    p = p * t2 + 9.9999749e-1
    s = t * p
    row = jax.lax.broadcasted_iota(jnp.int32, (N, N), 0)
    col = jax.lax.broadcasted_iota(jnp.int32, (N, N), 1)
    enm256 = jnp.where(row == col, jnp.zeros((), f32), s * (-128.0) + 128.0)

    iota_sp = jax.lax.broadcasted_iota(jnp.int32, (N, NUM_SPECIES), 1)
    onehot = (nf_ref[:, :] == iota_sp).astype(f32)
    h = (jnp.dot(onehot, se_ref[:, :], preferred_element_type=f32)
         + jnp.dot(gf_ref[:, :], Wt_ref[:, :], preferred_element_type=f32))

    f8 = jnp.float8_e4m3fn
    inv_scale = 1.0 / (AVG_NEIGH * 256.0 * 64.0)
    step = R_MAX / (NBASIS - 1)
    bf16 = jnp.bfloat16
    Gc = (jnp.exp(-d2p) * enm256).astype(bf16)
    F = jnp.exp(L * (2.0 * step)).astype(bf16)
    acc = jnp.zeros((N, HID), f32)
    for b in range(NBASIS):
        cb = jnp.bfloat16(np.exp(-(_MUS[b] ** 2)))
        rbb = (Gc * cb).astype(f8)
        rb_ref[b * N:(b + 1) * N, :] = rbb
        hb = (h * (Wrad_ref[b:b + 1, :] * 64.0)).astype(f8)
        acc = acc + jnp.dot(rbb, hb, preferred_element_type=f32)
        if b + 1 < NBASIS:
            Gc = Gc * F
    h = h + jnp.tanh(jnp.dot(acc * inv_scale,
                             Wupd_ref[0:HID, :], preferred_element_type=f32))

    acc = jnp.zeros((N, HID), f32)
    for b in range(NBASIS):
        rbb = rb_ref[b * N:(b + 1) * N, :]
        hb = (h * (Wrad_ref[NBASIS + b:NBASIS + b + 1, :] * 64.0)).astype(f8)
        acc = acc + jnp.dot(rbb, hb, preferred_element_type=f32)
    h = h + jnp.tanh(jnp.dot(acc * inv_scale,
                             Wupd_ref[HID:2 * HID, :], preferred_element_type=f32))

    scal = jnp.dot(jnp.tanh(jnp.dot(h, Wo1_ref[:, :], preferred_element_type=f32)),
                   Wo2_ref[:, :], preferred_element_type=f32)
    scal_row = jnp.transpose(scal)
    WmatT = jnp.where(row == col, jnp.zeros((), f32), scal_row * rinv)
    pos4 = jnp.concatenate([pos, jnp.ones((N, 1), f32)], axis=1)
    out4 = jnp.dot(WmatT, pos4, preferred_element_type=f32)
    vec = pos * out4[:, 3:4] - out4[:, 0:3]
    out_ref[:, :] = vec * (fs_ref[0, 0] * (1.0 / AVG_NEIGH))


def kernel(positions, node_features, global_features, species_embed, W_time,
           W_rad, W_upd, W_out1, W_out2, final_scaling):
    pos = positions.astype(jnp.float32)
    posT = pos.T
    nf = node_features.astype(jnp.int32).reshape(N, 1)
    gf = global_features.astype(jnp.float32).reshape(1, TDIM)
    Wrad2 = W_rad.astype(jnp.float32).reshape(NINTER * NBASIS, HID)
    Wupd2 = W_upd.astype(jnp.float32).reshape(NINTER * HID, HID)
    fs = final_scaling.astype(jnp.float32).reshape(1, 1)

    vmem = pl.BlockSpec(memory_space=pltpu.VMEM)
    return pl.pallas_call(
        _mace_body,
        out_shape=jax.ShapeDtypeStruct((N, 3), jnp.float32),
        in_specs=[vmem] * 11,
        out_specs=vmem,
        scratch_shapes=[pltpu.VMEM((NBASIS * N, N), jnp.float8_e4m3fn)],
    )(pos, posT, nf, gf, species_embed.astype(jnp.float32),
      W_time.astype(jnp.float32), Wrad2, Wupd2,
      W_out1.astype(jnp.float32), W_out2.astype(jnp.float32), fs)


# speedup vs baseline: 1.2554x; 1.0162x over previous
import jax
import jax.numpy as jnp
import numpy as np
from jax.experimental import pallas as pl
from jax.experimental.pallas import tpu as pltpu

N = 512
HID = 64
NUM_SPECIES = 10
TDIM = 32
NBASIS = 8
NINTER = 2
R_MAX = 5.0
AVG_NEIGH = 511.0

_MUS = [b * (R_MAX / (NBASIS - 1)) for b in range(NBASIS)]


def _mace_body(pos_ref, posT_ref, nf_ref, gf_ref, se_ref, Wt_ref, Wrad_ref,
               Wupd_ref, Wo1_ref, Wo2_ref, fs_ref, out_ref, rb_ref):
    f32 = jnp.float32
    pos = pos_ref[:, :]

    dk0 = posT_ref[0:1, :] - pos[:, 0:1]
    dk1 = posT_ref[1:2, :] - pos[:, 1:2]
    dk2 = posT_ref[2:3, :] - pos[:, 2:3]
    d2 = dk0 * dk0 + dk1 * dk1 + dk2 * dk2
    d2p = d2 + 1e-12
    rinv = jax.lax.rsqrt(d2p)
    L = d2p * rinv

    bf16 = jnp.bfloat16
    t = (((jnp.minimum(L * (1.0 / R_MAX), 1.0) - 0.5) * jnp.pi)
         ).astype(bf16)
    t2 = t * t
    p = t2 * jnp.bfloat16(-1.8447207e-4) + jnp.bfloat16(8.3095165e-3)
    p = p * t2 + jnp.bfloat16(-1.6665168e-1)
    p = p * t2 + jnp.bfloat16(9.9999749e-1)
    s = t * p
    row = jax.lax.broadcasted_iota(jnp.int32, (N, N), 0)
    col = jax.lax.broadcasted_iota(jnp.int32, (N, N), 1)
    enm256 = jnp.where(row == col, jnp.bfloat16(0.0),
                       s * jnp.bfloat16(-128.0) + jnp.bfloat16(128.0))

    iota_sp = jax.lax.broadcasted_iota(jnp.int32, (N, NUM_SPECIES), 1)
    onehot = (nf_ref[:, :] == iota_sp).astype(f32)
    h = (jnp.dot(onehot, se_ref[:, :], preferred_element_type=f32)
         + jnp.dot(gf_ref[:, :], Wt_ref[:, :], preferred_element_type=f32))

    f8 = jnp.float8_e4m3fn
    inv_scale = 1.0 / (AVG_NEIGH * 256.0 * 64.0)
    step = R_MAX / (NBASIS - 1)
    Gc = jnp.exp(-d2p).astype(bf16) * enm256
    F = jnp.exp(L * (2.0 * step)).astype(bf16)
    h_bf = h.astype(bf16)
    acc = jnp.zeros((N, HID), f32)
    for b in range(NBASIS):
        cb = jnp.bfloat16(np.exp(-(_MUS[b] ** 2)))
        rbb = (Gc * cb).astype(f8)
        rb_ref[:, b * N:(b + 1) * N] = rbb
        hb = (h_bf * (Wrad_ref[b:b + 1, :] * 64.0).astype(bf16)).astype(f8)
        acc = acc + jnp.dot(rbb, hb, preferred_element_type=f32)
        if b + 1 < NBASIS:
            Gc = Gc * F
    h = h + jnp.tanh(jnp.dot(acc * inv_scale,
                             Wupd_ref[0:HID, :], preferred_element_type=f32))

    h_bf = h.astype(bf16)
    hstack = jnp.concatenate(
        [(h_bf * (Wrad_ref[NBASIS + b:NBASIS + b + 1, :] * 64.0).astype(bf16)
          ).astype(f8) for b in range(NBASIS)], axis=0)
    acc = jnp.dot(rb_ref[:, :], hstack, preferred_element_type=f32)
    h = h + jnp.tanh(jnp.dot(acc * inv_scale,
                             Wupd_ref[HID:2 * HID, :], preferred_element_type=f32))

    scal = jnp.dot(jnp.tanh(jnp.dot(h, Wo1_ref[:, :], preferred_element_type=f32)),
                   Wo2_ref[:, :], preferred_element_type=f32)
    scal_row = jnp.transpose(scal)
    WmatT = jnp.where(row == col, jnp.zeros((), f32), scal_row * rinv)
    pos4 = jnp.concatenate([pos, jnp.ones((N, 1), f32)], axis=1)
    out4 = jnp.dot(WmatT, pos4, preferred_element_type=f32)
    vec = pos * out4[:, 3:4] - out4[:, 0:3]
    out_ref[:, :] = vec * (fs_ref[0, 0] * (1.0 / AVG_NEIGH))


def kernel(positions, node_features, global_features, species_embed, W_time,
           W_rad, W_upd, W_out1, W_out2, final_scaling):
    pos = positions.astype(jnp.float32)
    posT = pos.T
    nf = node_features.astype(jnp.int32).reshape(N, 1)
    gf = global_features.astype(jnp.float32).reshape(1, TDIM)
    Wrad2 = W_rad.astype(jnp.float32).reshape(NINTER * NBASIS, HID)
    Wupd2 = W_upd.astype(jnp.float32).reshape(NINTER * HID, HID)
    fs = final_scaling.astype(jnp.float32).reshape(1, 1)

    vmem = pl.BlockSpec(memory_space=pltpu.VMEM)
    return pl.pallas_call(
        _mace_body,
        out_shape=jax.ShapeDtypeStruct((N, 3), jnp.float32),
        in_specs=[vmem] * 11,
        out_specs=vmem,
        scratch_shapes=[pltpu.VMEM((N, NBASIS * N), jnp.float8_e4m3fn)],
    )(pos, posT, nf, gf, species_embed.astype(jnp.float32),
      W_time.astype(jnp.float32), Wrad2, Wupd2,
      W_out1.astype(jnp.float32), W_out2.astype(jnp.float32), fs)
